# Initial kernel scaffold; baseline (speedup 1.0000x reference)
import jax
import jax.numpy as jnp
from jax.experimental import pallas as pl


def kernel(pos, feat, edge_index):
    raise NotImplementedError("write your pallas kernel here")


# speedup vs baseline: 2.4842x; 2.4842x over previous
import functools

import jax
import jax.numpy as jnp
from jax import lax
from jax.experimental import pallas as pl
from jax.experimental.pallas import tpu as pltpu
from jax.experimental.pallas import tpu_sc as plsc

N_NODES = 10000
N_EDGES = 320000
D_FEAT = 128
D_OUT = D_FEAT + 3

NC = 2
NS = 16
NW = NC * NS
CHUNK = 320
NCHUNKS_TOTAL = N_EDGES // CHUNK
NGROUPS = CHUNK // 16


def _sc_kernel(pos_hbm, feat_hbm, src_hbm, dst_hbm, out_hbm,
               posv, srcv, dstv, fbuf, obuf, sem):
    wid = lax.axis_index("s") * NC + lax.axis_index("c")

    pltpu.sync_copy(pos_hbm, posv)

    lane = lax.iota(jnp.int32, 16)
    col_vecs = [lane + (3 + 16 * k) for k in range(8)]

    def chunk_body(t, carry):
        ci = wid + t * NW
        base = ci * CHUNK
        pltpu.sync_copy(src_hbm.at[pl.ds(base, CHUNK)], srcv)
        pltpu.sync_copy(dst_hbm.at[pl.ds(base, CHUNK)], dstv)
        pltpu.async_copy(feat_hbm.at[srcv], fbuf, sem).wait()

        def row_body(e, c2):
            ev16 = jnp.full((16,), e, dtype=jnp.int32)
            for k in range(8):
                v = fbuf[e, pl.ds(16 * k, 16)]
                plsc.store_scatter(obuf, [ev16, col_vecs[k]], v)
            return c2

        lax.fori_loop(0, CHUNK, row_body, 0)

        for j in range(NGROUPS):
            s3 = srcv[pl.ds(j * 16, 16)] * 3
            d3 = dstv[pl.ds(j * 16, 16)] * 3
            ev = lane + j * 16
            for c in range(3):
                ps = plsc.load_gather(posv, [s3 + c])
                pd = plsc.load_gather(posv, [d3 + c])
                plsc.store_scatter(obuf, [ev, jnp.full((16,), c, jnp.int32)],
                                   ps - pd)

        pltpu.sync_copy(obuf, out_hbm.at[pl.ds(base, CHUNK)])
        return carry

    nchunks = (NCHUNKS_TOTAL - 1 - wid) // NW + 1
    lax.fori_loop(0, nchunks, chunk_body, 0)


def kernel(pos, feat, edge_index):
    ei = edge_index.astype(jnp.int32)
    mesh = plsc.VectorSubcoreMesh(core_axis_name="c", subcore_axis_name="s")

    run = functools.partial(
        pl.kernel,
        mesh=mesh,
        compiler_params=pltpu.CompilerParams(
            needs_layout_passes=False, use_tc_tiling_on_sc=False),
        out_type=jax.ShapeDtypeStruct((N_EDGES, D_OUT), jnp.float32),
        scratch_types=[
            pltpu.VMEM((3 * N_NODES,), jnp.float32),
            pltpu.VMEM((CHUNK,), jnp.int32),
            pltpu.VMEM((CHUNK,), jnp.int32),
            pltpu.VMEM((CHUNK, D_FEAT), jnp.float32),
            pltpu.VMEM((CHUNK, D_OUT), jnp.float32),
            pltpu.SemaphoreType.DMA,
        ],
    )(_sc_kernel)
    return run(pos.reshape(-1), feat, ei[0], ei[1])


# speedup vs baseline: 3.3483x; 1.3478x over previous
import functools

import jax
import jax.numpy as jnp
from jax import lax
from jax.experimental import pallas as pl
from jax.experimental.pallas import tpu as pltpu
from jax.experimental.pallas import tpu_sc as plsc

N_NODES = 10000
N_EDGES = 320000
D_FEAT = 128
D_OUT = D_FEAT + 3

NC = 2
NS = 16
NW = NC * NS
CHUNK = 160
NCHUNKS_TOTAL = N_EDGES // CHUNK
NGROUPS = CHUNK // 16


def _sc_kernel(pos_hbm, feat_hbm, src_hbm, dst_hbm, out_hbm,
               posv, srcv, dstv, fbuf, obuf, sem):
    wid = lax.axis_index("s") * NC + lax.axis_index("c")

    pltpu.sync_copy(pos_hbm, posv)

    lane = lax.iota(jnp.int32, 16)
    col_vecs = [lane + (3 + 16 * k) for k in range(8)]

    def chunk_body(t, carry):
        ci = wid + t * NW
        base = ci * CHUNK
        pltpu.sync_copy(src_hbm.at[pl.ds(base, CHUNK)], srcv)
        pltpu.sync_copy(dst_hbm.at[pl.ds(base, CHUNK)], dstv)
        pltpu.async_copy(feat_hbm.at[srcv], fbuf, sem).wait()

        def row_body(e, c2):
            ev16 = jnp.full((16,), e, dtype=jnp.int32)
            for k in range(8):
                v = fbuf[e, pl.ds(16 * k, 16)]
                plsc.store_scatter(obuf, [ev16, col_vecs[k]], v)
            return c2

        lax.fori_loop(0, CHUNK, row_body, 0)

        for j in range(NGROUPS):
            s3 = srcv[pl.ds(j * 16, 16)] * 3
            d3 = dstv[pl.ds(j * 16, 16)] * 3
            ev = lane + j * 16
            for c in range(3):
                ps = plsc.load_gather(posv, [s3 + c])
                pd = plsc.load_gather(posv, [d3 + c])
                plsc.store_scatter(obuf, [ev, jnp.full((16,), c, jnp.int32)],
                                   ps - pd)

        pltpu.sync_copy(obuf, out_hbm.at[pl.ds(base, CHUNK)])
        return carry

    nchunks = (NCHUNKS_TOTAL - 1 - wid) // NW + 1
    lax.fori_loop(0, nchunks, chunk_body, 0)


def kernel(pos, feat, edge_index):
    ei = edge_index.astype(jnp.int32)
    mesh = plsc.VectorSubcoreMesh(core_axis_name="c", subcore_axis_name="s")

    run = functools.partial(
        pl.kernel,
        mesh=mesh,
        compiler_params=pltpu.CompilerParams(
            needs_layout_passes=False, use_tc_tiling_on_sc=True),
        out_type=jax.ShapeDtypeStruct((N_EDGES, D_OUT), jnp.float32),
        scratch_types=[
            pltpu.VMEM((3 * N_NODES,), jnp.float32),
            pltpu.VMEM((CHUNK,), jnp.int32),
            pltpu.VMEM((CHUNK,), jnp.int32),
            pltpu.VMEM((CHUNK, D_FEAT), jnp.float32),
            pltpu.VMEM((CHUNK, D_OUT), jnp.float32),
            pltpu.SemaphoreType.DMA,
        ],
    )(_sc_kernel)
    return run(pos.reshape(-1), feat, ei[0], ei[1])
